# baseline (speedup 1.0000x reference)
import functools

import jax
import jax.numpy as jnp
from jax import lax
from jax.experimental import pallas as pl
from jax.experimental.pallas import tpu as pltpu
from jax.experimental.pallas import tpu_sc as plsc

NC = 2
NS = 16
NW = NC * NS
LANES = 16
EPS = 1e-5
NBUF = 4


@functools.partial(jax.jit, static_argnums=(5, 6))
def _sc_embed_ln(ids3d, tok_table, pos_table, gamma, beta, seq, chunk):
    _, n_chunks, _ = ids3d.shape
    rows = NW * n_chunks * chunk
    hidden = tok_table.shape[1]
    rows_pw = rows // NW
    nslice = hidden // LANES
    mesh = plsc.VectorSubcoreMesh(
        core_axis_name="c", subcore_axis_name="s", num_cores=NC, num_subcores=NS
    )

    @functools.partial(
        pl.kernel,
        out_type=jax.ShapeDtypeStruct((rows, hidden), jnp.float32),
        mesh=mesh,
        compiler_params=pltpu.CompilerParams(needs_layout_passes=False),
        scratch_types=[
            pltpu.VMEM((n_chunks, chunk), jnp.int32),
            pltpu.VMEM((NBUF * chunk, hidden), jnp.float32),
            pltpu.VMEM((NBUF * chunk, hidden), jnp.float32),
            pltpu.VMEM((hidden,), jnp.float32),
            pltpu.VMEM((hidden,), jnp.float32),
            pltpu.SemaphoreType.DMA((NBUF,)),
            pltpu.SemaphoreType.DMA((NBUF,)),
            pltpu.SemaphoreType.DMA((NBUF,)),
        ],
    )
    def run(ids_hbm, tok_hbm, pos_hbm, gamma_hbm, beta_hbm, out_hbm,
            idx_v, buf_v, pos_v, g_v, b_v, gsem, psem, osem):
        cid = lax.axis_index("c")
        sid = lax.axis_index("s")
        wid = sid * NC + cid
        row0 = wid * rows_pw
        s0 = row0 % seq

        pltpu.sync_copy(ids_hbm.at[wid], idx_v)
        pltpu.sync_copy(gamma_hbm, g_v)
        pltpu.sync_copy(beta_hbm, b_v)

        def issue_in(c):
            q = lax.rem(c, NBUF)
            sl = pl.ds(q * chunk, chunk)
            pltpu.async_copy(tok_hbm.at[idx_v.at[c]], buf_v.at[sl], gsem.at[q])
            pltpu.async_copy(
                pos_hbm.at[pl.ds(s0 + c * chunk, chunk)], pos_v.at[sl], psem.at[q]
            )

        def wait_in(q):
            sl = pl.ds(q * chunk, chunk)
            pltpu.make_async_copy(
                tok_hbm.at[pl.ds(0, chunk)], buf_v.at[sl], gsem.at[q]
            ).wait()
            pltpu.make_async_copy(
                pos_hbm.at[pl.ds(0, chunk)], pos_v.at[sl], psem.at[q]
            ).wait()

        def wait_out(q):
            pltpu.make_async_copy(
                buf_v.at[pl.ds(q * chunk, chunk)],
                out_hbm.at[pl.ds(0, chunk)],
                osem.at[q],
            ).wait()

        def compute(base):
            @plsc.parallel_loop(0, chunk, 1, unroll=4)
            def row_body(rr):
                r = base + rr
                acc = jnp.zeros((LANES,), jnp.float32)
                acc2 = jnp.zeros((LANES,), jnp.float32)
                for s in range(nslice):
                    sl = pl.ds(s * LANES, LANES)
                    x = buf_v[r, sl] + pos_v[r, sl]
                    buf_v[r, sl] = x
                    acc = acc + x
                    acc2 = acc2 + x * x
                tot = plsc.cumsum(acc)[LANES - 1]
                tot2 = plsc.cumsum(acc2)[LANES - 1]
                rhidden = jnp.float32(1.0 / hidden)
                mean = tot * rhidden
                var = tot2 * rhidden - mean * mean
                v = var + EPS
                i = lax.bitcast_convert_type(v, jnp.int32)
                y = lax.bitcast_convert_type(
                    jnp.int32(0x5F3759DF) - (i >> 1), jnp.float32
                )
                for _ in range(3):
                    y = y * (1.5 - 0.5 * v * y * y)
                scale = y
                shift = -mean * y
                for s in range(nslice):
                    sl = pl.ds(s * LANES, LANES)
                    x = buf_v[r, sl]
                    buf_v[r, sl] = (x * scale + shift) * g_v[sl] + b_v[sl]

        for c in range(NBUF - 1):
            issue_in(c)

        def chunk_body(c, carry):
            q = lax.rem(c, NBUF)
            wait_in(q)
            compute(q * chunk)
            pltpu.async_copy(
                buf_v.at[pl.ds(q * chunk, chunk)],
                out_hbm.at[pl.ds(row0 + c * chunk, chunk)],
                osem.at[q],
            )
            nxt = c + NBUF - 1
            qn = lax.rem(nxt, NBUF)

            @pl.when(nxt < n_chunks)
            def _():
                @pl.when(c >= 1)
                def _():
                    wait_out(qn)

                issue_in(nxt)

            return carry

        lax.fori_loop(0, n_chunks, chunk_body, 0)

        for q in range(NBUF):
            wait_out(q)

    return run(ids3d, tok_table, pos_table, gamma, beta)


def kernel(token_ids, tok_table, pos_table, gamma, beta):
    batch, seq = token_ids.shape
    hidden = tok_table.shape[1]
    chunk = 16
    rows_pw = batch * seq // NW
    ids3d = token_ids.reshape(NW, rows_pw // chunk, chunk).astype(jnp.int32)
    out = _sc_embed_ln(ids3d, tok_table, pos_table, gamma, beta, seq, chunk)
    return out.reshape(batch, seq, hidden)


# speedup vs baseline: 4.8774x; 4.8774x over previous
import functools

import jax
import jax.numpy as jnp
from jax import lax
from jax.experimental import pallas as pl
from jax.experimental.pallas import tpu as pltpu
from jax.experimental.pallas import tpu_sc as plsc

NC = 2
NS = 16
NW = NC * NS
LANES = 16
EPS = 1e-5
NBUF = 4


@functools.partial(jax.jit, static_argnums=(5, 6))
def _sc_embed_ln(ids3d, tok_table, pos_table, gamma, beta, seq, chunk):
    _, n_chunks, _ = ids3d.shape
    rows = NW * n_chunks * chunk
    hidden = tok_table.shape[1]
    rows_pw = rows // NW
    nslice = hidden // LANES
    mesh = plsc.VectorSubcoreMesh(
        core_axis_name="c", subcore_axis_name="s", num_cores=NC, num_subcores=NS
    )

    @functools.partial(
        pl.kernel,
        out_type=jax.ShapeDtypeStruct((rows, hidden), jnp.float32),
        mesh=mesh,
        compiler_params=pltpu.CompilerParams(needs_layout_passes=False),
        scratch_types=[
            pltpu.VMEM((n_chunks, chunk), jnp.int32),
            pltpu.VMEM((NBUF * chunk, hidden), jnp.float32),
            pltpu.VMEM((NBUF * chunk, hidden), jnp.float32),
            pltpu.VMEM((hidden,), jnp.float32),
            pltpu.VMEM((hidden,), jnp.float32),
            pltpu.VMEM((chunk, LANES), jnp.float32),
            pltpu.VMEM((chunk, LANES), jnp.float32),
            pltpu.SemaphoreType.DMA((NBUF,)),
            pltpu.SemaphoreType.DMA((NBUF,)),
            pltpu.SemaphoreType.DMA((NBUF,)),
        ],
    )
    def run(ids_hbm, tok_hbm, pos_hbm, gamma_hbm, beta_hbm, out_hbm,
            idx_v, buf_v, pos_v, g_v, b_v, sc_v, sh_v, gsem, psem, osem):
        cid = lax.axis_index("c")
        sid = lax.axis_index("s")
        wid = sid * NC + cid
        row0 = wid * rows_pw
        s0 = row0 % seq

        pltpu.sync_copy(ids_hbm.at[wid], idx_v)
        pltpu.sync_copy(gamma_hbm, g_v)
        pltpu.sync_copy(beta_hbm, b_v)

        def issue_in(c):
            q = lax.rem(c, NBUF)
            sl = pl.ds(q * chunk, chunk)
            pltpu.async_copy(tok_hbm.at[idx_v.at[c]], buf_v.at[sl], gsem.at[q])
            pltpu.async_copy(
                pos_hbm.at[pl.ds(s0 + c * chunk, chunk)], pos_v.at[sl], psem.at[q]
            )

        def wait_in(q):
            sl = pl.ds(q * chunk, chunk)
            pltpu.make_async_copy(
                tok_hbm.at[pl.ds(0, chunk)], buf_v.at[sl], gsem.at[q]
            ).wait()
            pltpu.make_async_copy(
                pos_hbm.at[pl.ds(0, chunk)], pos_v.at[sl], psem.at[q]
            ).wait()

        def wait_out(q):
            pltpu.make_async_copy(
                buf_v.at[pl.ds(q * chunk, chunk)],
                out_hbm.at[pl.ds(0, chunk)],
                osem.at[q],
            ).wait()

        nsec = 3
        sec_slices = nslice // nsec

        def compute(base):
            @plsc.parallel_loop(0, chunk, 1, unroll=2)
            def row_body(rr):
                r = base + rr
                acc = jnp.zeros((LANES,), jnp.float32)
                acc2 = jnp.zeros((LANES,), jnp.float32)
                for s in range(nslice):
                    sl = pl.ds(s * LANES, LANES)
                    x = buf_v[r, sl] + pos_v[r, sl]
                    buf_v[r, sl] = x
                    acc = acc + x
                    acc2 = acc2 + x * x
                tot = plsc.cumsum(acc)[LANES - 1]
                tot2 = plsc.cumsum(acc2)[LANES - 1]
                rhidden = jnp.float32(1.0 / hidden)
                mean = tot * rhidden
                var = tot2 * rhidden - mean * mean
                v = var + EPS
                i = lax.bitcast_convert_type(v, jnp.int32)
                y = lax.bitcast_convert_type(
                    jnp.int32(0x5F3759DF) - (i >> 1), jnp.float32
                )
                for _ in range(3):
                    y = y * (1.5 - 0.5 * v * y * y)
                sc_v[rr, :] = jnp.full((LANES,), y)
                sh_v[rr, :] = jnp.full((LANES,), -mean * y)

            for sec in range(nsec):
                gs = [g_v[pl.ds((sec * sec_slices + s) * LANES, LANES)]
                      for s in range(sec_slices)]
                bs = [b_v[pl.ds((sec * sec_slices + s) * LANES, LANES)]
                      for s in range(sec_slices)]

                @plsc.parallel_loop(0, chunk, 1, unroll=2)
                def aff_body(rr):
                    r = base + rr
                    scale = sc_v[rr, :]
                    shift = sh_v[rr, :]
                    for s in range(sec_slices):
                        sl = pl.ds((sec * sec_slices + s) * LANES, LANES)
                        x = buf_v[r, sl]
                        buf_v[r, sl] = (x * scale + shift) * gs[s] + bs[s]

        for c in range(NBUF - 1):
            issue_in(c)

        def chunk_body(c, carry):
            q = lax.rem(c, NBUF)
            wait_in(q)
            compute(q * chunk)
            pltpu.async_copy(
                buf_v.at[pl.ds(q * chunk, chunk)],
                out_hbm.at[pl.ds(row0 + c * chunk, chunk)],
                osem.at[q],
            )
            nxt = c + NBUF - 1
            qn = lax.rem(nxt, NBUF)

            @pl.when(nxt < n_chunks)
            def _():
                @pl.when(c >= 1)
                def _():
                    wait_out(qn)

                issue_in(nxt)

            return carry

        lax.fori_loop(0, n_chunks, chunk_body, 0)

        for q in range(NBUF):
            wait_out(q)

    return run(ids3d, tok_table, pos_table, gamma, beta)


def kernel(token_ids, tok_table, pos_table, gamma, beta):
    batch, seq = token_ids.shape
    hidden = tok_table.shape[1]
    chunk = 16
    rows_pw = batch * seq // NW
    ids3d = token_ids.reshape(NW, rows_pw // chunk, chunk).astype(jnp.int32)
    out = _sc_embed_ln(ids3d, tok_table, pos_table, gamma, beta, seq, chunk)
    return out.reshape(batch, seq, hidden)
